# baseline (speedup 1.0000x reference)
import functools

import jax
import jax.numpy as jnp
from jax import lax
from jax.experimental import pallas as pl
from jax.experimental.pallas import tpu as pltpu
from jax.experimental.pallas import tpu_sc as plsc

N, E, D, DE, H = 10000, 320000, 128, 4, 128

NC, NS = 2, 16
NW = NC * NS
EPW = E // NW
CHUNK = 40
NCHUNK = EPW // CHUNK
NBUF = 4
GROUPS = NCHUNK // NBUF
TAIL = NCHUNK - GROUPS * NBUF
RPT = 624
RTAIL = N - NS * RPT


def _y_body(x_ref, w_ref, o_ref):
    o_ref[...] = jnp.dot(x_ref[...], w_ref[..., :D, :],
                         preferred_element_type=jnp.float32)


def _prep_body(ea_ref, w_ref, b_ref, ei_ref, t_ref, s_ref, d_ref):
    tb = (jnp.dot(ea_ref[...], w_ref[..., D:, :],
                  preferred_element_type=jnp.float32) + b_ref[...])
    lo = jax.lax.bitcast_convert_type(
        tb[:, :H // 2].astype(jnp.bfloat16), jnp.uint16).astype(jnp.uint32)
    hi = jax.lax.bitcast_convert_type(
        tb[:, H // 2:].astype(jnp.bfloat16), jnp.uint16).astype(jnp.uint32)
    t_ref[...] = jax.lax.bitcast_convert_type(lo | (hi << 16), jnp.int32)
    i = pl.program_id(0)
    sl = pl.ds(i * _PREP_BE, _PREP_BE)
    s_ref[sl] = ei_ref[0, sl]
    d_ref[sl] = ei_ref[1, sl]


def _upd_body(x_ref, a_ref, w_ref, b_ref, o_ref):
    agg = a_ref[0] + a_ref[1]
    h = (jnp.dot(x_ref[...], w_ref[..., :D, :],
                 preferred_element_type=jnp.float32)
         + jnp.dot(agg, w_ref[..., D:, :],
                   preferred_element_type=jnp.float32)
         + b_ref[...])
    o_ref[...] = jnp.maximum(h, 0.0) + x_ref[...]


def _tc_y(x, wm):
    bn = 2000
    return pl.pallas_call(
        _y_body,
        grid=(N // bn,),
        in_specs=[pl.BlockSpec((bn, D), lambda i: (i, 0)),
                  pl.BlockSpec((D + DE, H), lambda i: (0, 0))],
        out_specs=pl.BlockSpec((bn, H), lambda i: (i, 0)),
        out_shape=jax.ShapeDtypeStruct((N, H), jnp.float32),
    )(x, wm)


_PREP_BE = 3200


def _tc_prep(ea, wm, bm, ei):
    be = _PREP_BE
    return pl.pallas_call(
        _prep_body,
        grid=(E // be,),
        in_specs=[pl.BlockSpec((be, DE), lambda i: (i, 0)),
                  pl.BlockSpec((D + DE, H), lambda i: (0, 0)),
                  pl.BlockSpec((1, H), lambda i: (0, 0)),
                  pl.BlockSpec((2, E), lambda i: (0, 0))],
        out_specs=[pl.BlockSpec((be, H // 2), lambda i: (i, 0)),
                   pl.BlockSpec((E,), lambda i: (0,)),
                   pl.BlockSpec((E,), lambda i: (0,))],
        out_shape=[jax.ShapeDtypeStruct((E, H // 2), jnp.int32),
                   jax.ShapeDtypeStruct((E,), jnp.int32),
                   jax.ShapeDtypeStruct((E,), jnp.int32)],
    )(ea, wm, bm, ei)


def _tc_update(x, agg_p, wu, bu):
    bn = 2000
    return pl.pallas_call(
        _upd_body,
        grid=(N // bn,),
        in_specs=[pl.BlockSpec((bn, D), lambda i: (i, 0)),
                  pl.BlockSpec((2, bn, H), lambda i: (0, i, 0)),
                  pl.BlockSpec((D + H, D), lambda i: (0, 0)),
                  pl.BlockSpec((1, D), lambda i: (0, 0))],
        out_specs=pl.BlockSpec((bn, D), lambda i: (i, 0)),
        out_shape=jax.ShapeDtypeStruct((N, D), jnp.float32),
    )(x, agg_p, wu, bu)


def _sc_edges(src, dst, t, y):
    mesh = plsc.VectorSubcoreMesh(core_axis_name="c", subcore_axis_name="s")

    scratch = (
        [pltpu.VMEM((CHUNK,), jnp.int32) for _ in range(2 * NBUF)]
        + [pltpu.VMEM((NBUF * CHUNK, H), jnp.float32)]
        + [pltpu.VMEM((NBUF * CHUNK, H // 2), jnp.int32)]
        + [pltpu.SemaphoreType.DMA for _ in range(3 * NBUF)]
        + [pltpu.VMEM_SHARED((N, H), jnp.float32)]
    )

    @functools.partial(
        pl.kernel,
        mesh=mesh,
        out_type=jax.ShapeDtypeStruct((NC, N, H), jnp.float32),
        scratch_types=scratch,
        compiler_params=pltpu.CompilerParams(needs_layout_passes=False),
    )
    def k(src_hbm, dst_hbm, t_hbm, y_hbm, out_hbm, *rest):
        idx_s = rest[0:NBUF]
        idx_d = rest[NBUF:2 * NBUF]
        mbuf = rest[2 * NBUF]
        tbuf = rest[2 * NBUF + 1]
        lsem = rest[2 * NBUF + 2:2 * NBUF + 2 + NBUF]
        gsem = rest[2 * NBUF + 2 + NBUF:2 * NBUF + 2 + 2 * NBUF]
        ssem = rest[2 * NBUF + 2 + 2 * NBUF:2 * NBUF + 2 + 3 * NBUF]
        acc = rest[-1]

        cid = lax.axis_index("c")
        sid = lax.axis_index("s")
        wid = cid * NS + sid
        ebase = wid * EPW
        rbase = sid * RPT

        def msl(j):
            return mbuf.at[pl.ds(j * CHUNK, CHUNK)]

        @plsc.parallel_loop(0, NBUF * CHUNK, unroll=4)
        def _(i):
            for q in range(H // 16):
                mbuf[i, pl.ds(q * 16, 16)] = jnp.zeros((16,), jnp.float32)
        nz = NBUF * CHUNK
        for off in range(0, RPT, nz):
            sz = min(nz, RPT - off)
            pltpu.sync_copy(mbuf.at[pl.ds(0, sz)],
                            acc.at[pl.ds(rbase + off, sz)])

        @pl.when(sid == 0)
        def _():
            pltpu.sync_copy(mbuf.at[pl.ds(0, RTAIL)],
                            acc.at[pl.ds(NS * RPT, RTAIL)])
        plsc.subcore_barrier()

        def tsl(j):
            return tbuf.at[pl.ds(j * CHUNK, CHUNK)]

        def issue_loads(c, j):
            base = ebase + c * CHUNK
            pltpu.async_copy(src_hbm.at[pl.ds(base, CHUNK)], idx_s[j], lsem[j])
            pltpu.async_copy(dst_hbm.at[pl.ds(base, CHUNK)], idx_d[j], lsem[j])
            pltpu.async_copy(t_hbm.at[pl.ds(base, CHUNK)], tsl(j), lsem[j])

        def wait_loads(c, j):
            base = ebase + c * CHUNK
            pltpu.make_async_copy(src_hbm.at[pl.ds(base, CHUNK)], idx_s[j],
                                  lsem[j]).wait()
            pltpu.make_async_copy(dst_hbm.at[pl.ds(base, CHUNK)], idx_d[j],
                                  lsem[j]).wait()
            pltpu.make_async_copy(t_hbm.at[pl.ds(base, CHUNK)], tsl(j),
                                  lsem[j]).wait()

        def edge_math(j):
            br = j * CHUNK

            @plsc.parallel_loop(0, CHUNK, unroll=2)
            def _(e):
                row = br + e
                for q in range(H // 32):
                    tv = plsc.bitcast(tbuf[row, pl.ds(q * 16, 16)],
                                      jnp.bfloat16)
                    tlo, thi = plsc.unpack(tv,
                                           format=plsc.PackFormat.INTERLEAVED)
                    slo = pl.ds(q * 16, 16)
                    shi = pl.ds(H // 2 + q * 16, 16)
                    mbuf[row, slo] = jnp.maximum(mbuf[row, slo] + tlo, 0.0)
                    mbuf[row, shi] = jnp.maximum(mbuf[row, shi] + thi, 0.0)

        for j in range(NBUF):
            issue_loads(j, j)

        def group(i, _):
            gathers = []
            for j in range(NBUF):
                wait_loads(i * NBUF + j, j)
                gathers.append(
                    pltpu.async_copy(y_hbm.at[idx_s[j]], msl(j), gsem[j]))
            scatters = []
            for j in range(NBUF):
                gathers[j].wait()
                edge_math(j)
                scatters.append(
                    pltpu.async_copy(msl(j), acc.at[idx_d[j]], ssem[j],
                                     add=True))
            for j in range(NBUF):
                scatters[j].wait()

                @pl.when(i < GROUPS - 1)
                def _(j=j):
                    issue_loads((i + 1) * NBUF + j, j)
            return 0
        lax.fori_loop(0, GROUPS, group, 0)

        for j in range(TAIL):
            issue_loads(GROUPS * NBUF + j, j)
        for j in range(TAIL):
            wait_loads(GROUPS * NBUF + j, j)
            pltpu.async_copy(y_hbm.at[idx_s[j]], msl(j), gsem[j]).wait()
            edge_math(j)
            pltpu.async_copy(msl(j), acc.at[idx_d[j]], ssem[j],
                             add=True).wait()

        plsc.subcore_barrier()
        pltpu.sync_copy(acc.at[pl.ds(rbase, RPT)],
                        out_hbm.at[cid, pl.ds(rbase, RPT)])

        @pl.when(sid == 0)
        def _():
            pltpu.sync_copy(acc.at[pl.ds(NS * RPT, RTAIL)],
                            out_hbm.at[cid, pl.ds(NS * RPT, RTAIL)])

    return k(src, dst, t, y)


def kernel(x, edge_index, edge_attr, W_msg, b_msg, W_upd, b_upd):
    bm = b_msg.reshape(1, H)
    bu = b_upd.reshape(1, D)

    y = _tc_y(x, W_msg)
    t, src, dst = _tc_prep(edge_attr, W_msg, bm, edge_index)
    agg_p = _sc_edges(src, dst, t, y)
    return _tc_update(x, agg_p, W_upd, bu)


# speedup vs baseline: 1.1634x; 1.1634x over previous
import functools

import jax
import jax.numpy as jnp
from jax import lax
from jax.experimental import pallas as pl
from jax.experimental.pallas import tpu as pltpu
from jax.experimental.pallas import tpu_sc as plsc

N, E, D, DE, H = 10000, 320000, 128, 4, 128

NC, NS = 2, 16
NW = NC * NS
EPW = E // NW
CHUNK = 80
NCHUNK = EPW // CHUNK
NBUF = 4
GROUPS = (NCHUNK - 1) // NBUF
RPT = 624
RTAIL = N - NS * RPT


def _y_body(x_ref, w_ref, o_ref):
    o_ref[...] = jnp.dot(x_ref[...], w_ref[..., :D, :],
                         preferred_element_type=jnp.float32)


def _prep_body(ea_ref, w_ref, b_ref, ei_ref, t_ref, s_ref, d_ref):
    t_ref[...] = (jnp.dot(ea_ref[...], w_ref[..., D:, :],
                          preferred_element_type=jnp.float32) + b_ref[...])
    i = pl.program_id(0)
    sl = pl.ds(i * _PREP_BE, _PREP_BE)
    s_ref[sl] = ei_ref[0, sl]
    d_ref[sl] = ei_ref[1, sl]


def _upd_body(x_ref, a_ref, w_ref, b_ref, o_ref):
    agg = a_ref[0] + a_ref[1]
    h = (jnp.dot(x_ref[...], w_ref[..., :D, :],
                 preferred_element_type=jnp.float32)
         + jnp.dot(agg, w_ref[..., D:, :],
                   preferred_element_type=jnp.float32)
         + b_ref[...])
    o_ref[...] = jnp.maximum(h, 0.0) + x_ref[...]


def _tc_y(x, wm):
    bn = 2000
    return pl.pallas_call(
        _y_body,
        grid=(N // bn,),
        in_specs=[pl.BlockSpec((bn, D), lambda i: (i, 0)),
                  pl.BlockSpec((D + DE, H), lambda i: (0, 0))],
        out_specs=pl.BlockSpec((bn, H), lambda i: (i, 0)),
        out_shape=jax.ShapeDtypeStruct((N, H), jnp.float32),
    )(x, wm)


_PREP_BE = 3200


def _tc_prep(ea, wm, bm, ei):
    be = _PREP_BE
    return pl.pallas_call(
        _prep_body,
        grid=(E // be,),
        in_specs=[pl.BlockSpec((be, DE), lambda i: (i, 0)),
                  pl.BlockSpec((D + DE, H), lambda i: (0, 0)),
                  pl.BlockSpec((1, H), lambda i: (0, 0)),
                  pl.BlockSpec((2, E), lambda i: (0, 0))],
        out_specs=[pl.BlockSpec((be, H), lambda i: (i, 0)),
                   pl.BlockSpec((E,), lambda i: (0,)),
                   pl.BlockSpec((E,), lambda i: (0,))],
        out_shape=[jax.ShapeDtypeStruct((E, H), jnp.float32),
                   jax.ShapeDtypeStruct((E,), jnp.int32),
                   jax.ShapeDtypeStruct((E,), jnp.int32)],
    )(ea, wm, bm, ei)


def _tc_update(x, agg_p, wu, bu):
    bn = 2000
    return pl.pallas_call(
        _upd_body,
        grid=(N // bn,),
        in_specs=[pl.BlockSpec((bn, D), lambda i: (i, 0)),
                  pl.BlockSpec((2, bn, H), lambda i: (0, i, 0)),
                  pl.BlockSpec((D + H, D), lambda i: (0, 0)),
                  pl.BlockSpec((1, D), lambda i: (0, 0))],
        out_specs=pl.BlockSpec((bn, D), lambda i: (i, 0)),
        out_shape=jax.ShapeDtypeStruct((N, D), jnp.float32),
    )(x, agg_p, wu, bu)


def _sc_edges(src, dst, t, y):
    mesh = plsc.VectorSubcoreMesh(core_axis_name="c", subcore_axis_name="s")

    scratch = (
        [pltpu.VMEM((CHUNK,), jnp.int32) for _ in range(2 * NBUF)]
        + [pltpu.VMEM((NBUF * CHUNK, H), jnp.float32)]
        + [pltpu.SemaphoreType.DMA for _ in range(3 * NBUF)]
        + [pltpu.VMEM_SHARED((N, H), jnp.float32)]
    )

    @functools.partial(
        pl.kernel,
        mesh=mesh,
        out_type=jax.ShapeDtypeStruct((NC, N, H), jnp.float32),
        scratch_types=scratch,
    )
    def k(src_hbm, dst_hbm, t_hbm, y_hbm, out_hbm, *rest):
        idx_s = rest[0:NBUF]
        idx_d = rest[NBUF:2 * NBUF]
        mbuf = rest[2 * NBUF]
        lsem = rest[2 * NBUF + 1:2 * NBUF + 1 + NBUF]
        gsem = rest[2 * NBUF + 1 + NBUF:2 * NBUF + 1 + 2 * NBUF]
        ssem = rest[2 * NBUF + 1 + 2 * NBUF:2 * NBUF + 1 + 3 * NBUF]
        acc = rest[-1]

        cid = lax.axis_index("c")
        sid = lax.axis_index("s")
        wid = cid * NS + sid
        ebase = wid * EPW
        rbase = sid * RPT

        def msl(j):
            return mbuf.at[pl.ds(j * CHUNK, CHUNK)]

        @plsc.parallel_loop(0, NBUF * CHUNK, unroll=4)
        def _(i):
            for q in range(H // 16):
                mbuf[i, pl.ds(q * 16, 16)] = jnp.zeros((16,), jnp.float32)
        pltpu.sync_copy(mbuf.at[pl.ds(0, NBUF * CHUNK)],
                        acc.at[pl.ds(rbase, NBUF * CHUNK)])
        pltpu.sync_copy(mbuf.at[pl.ds(0, RPT - NBUF * CHUNK)],
                        acc.at[pl.ds(rbase + NBUF * CHUNK,
                                     RPT - NBUF * CHUNK)])

        @pl.when(sid == 0)
        def _():
            pltpu.sync_copy(mbuf.at[pl.ds(0, RTAIL)],
                            acc.at[pl.ds(NS * RPT, RTAIL)])
        plsc.subcore_barrier()

        def issue_loads(c, j):
            base = ebase + c * CHUNK
            pltpu.async_copy(src_hbm.at[pl.ds(base, CHUNK)], idx_s[j], lsem[j])
            pltpu.async_copy(dst_hbm.at[pl.ds(base, CHUNK)], idx_d[j], lsem[j])
            pltpu.async_copy(t_hbm.at[pl.ds(base, CHUNK)], msl(j), lsem[j])

        def wait_loads(c, j):
            base = ebase + c * CHUNK
            pltpu.make_async_copy(src_hbm.at[pl.ds(base, CHUNK)], idx_s[j],
                                  lsem[j]).wait()
            pltpu.make_async_copy(dst_hbm.at[pl.ds(base, CHUNK)], idx_d[j],
                                  lsem[j]).wait()
            pltpu.make_async_copy(t_hbm.at[pl.ds(base, CHUNK)], msl(j),
                                  lsem[j]).wait()

        def relu_buf(j):
            br = j * CHUNK

            @plsc.parallel_loop(0, CHUNK, unroll=4)
            def _(e):
                for q in range(H // 16):
                    sl = pl.ds(q * 16, 16)
                    mbuf[br + e, sl] = jnp.maximum(mbuf[br + e, sl], 0.0)

        for j in range(NBUF):
            issue_loads(j, j)

        def group(i, _):
            gathers = []
            for j in range(NBUF):
                wait_loads(i * NBUF + j, j)
                gathers.append(
                    pltpu.async_copy(y_hbm.at[idx_s[j]], msl(j), gsem[j],
                                     add=True))
            scatters = []
            for j in range(NBUF):
                gathers[j].wait()
                relu_buf(j)
                scatters.append(
                    pltpu.async_copy(msl(j), acc.at[idx_d[j]], ssem[j],
                                     add=True))
            for j in range(NBUF):
                scatters[j].wait()

                @pl.when(i < GROUPS - 1)
                def _(j=j):
                    issue_loads((i + 1) * NBUF + j, j)

                if j == 0:
                    @pl.when(i == GROUPS - 1)
                    def _():
                        issue_loads(NCHUNK - 1, 0)
            return 0
        lax.fori_loop(0, GROUPS, group, 0)

        wait_loads(NCHUNK - 1, 0)
        pltpu.async_copy(y_hbm.at[idx_s[0]], msl(0), gsem[0], add=True).wait()
        relu_buf(0)
        pltpu.async_copy(msl(0), acc.at[idx_d[0]], ssem[0], add=True).wait()

        plsc.subcore_barrier()
        pltpu.sync_copy(acc.at[pl.ds(rbase, RPT)],
                        out_hbm.at[cid, pl.ds(rbase, RPT)])

        @pl.when(sid == 0)
        def _():
            pltpu.sync_copy(acc.at[pl.ds(NS * RPT, RTAIL)],
                            out_hbm.at[cid, pl.ds(NS * RPT, RTAIL)])

    return k(src, dst, t, y)


def kernel(x, edge_index, edge_attr, W_msg, b_msg, W_upd, b_upd):
    bm = b_msg.reshape(1, H)
    bu = b_upd.reshape(1, D)

    y = _tc_y(x, W_msg)
    t, src, dst = _tc_prep(edge_attr, W_msg, bm, edge_index)
    agg_p = _sc_edges(src, dst, t, y)
    return _tc_update(x, agg_p, W_upd, bu)


# speedup vs baseline: 1.2364x; 1.0627x over previous
import functools

import jax
import jax.numpy as jnp
from jax import lax
from jax.experimental import pallas as pl
from jax.experimental.pallas import tpu as pltpu
from jax.experimental.pallas import tpu_sc as plsc

N, E, D, DE, H = 10000, 320000, 128, 4, 128

NC, NS = 2, 16
NW = NC * NS
EPW = E // NW
CHUNK = 80
NCHUNK = EPW // CHUNK
NBUF = 4
GROUPS = (NCHUNK - 1) // NBUF
RPT = 624
RTAIL = N - NS * RPT


def _y_body(x_ref, w_ref, o_ref):
    o_ref[...] = jnp.dot(x_ref[...], w_ref[..., :D, :],
                         preferred_element_type=jnp.float32)


def _prep_body(ea_ref, w_ref, b_ref, ei_ref, t_ref, s_ref, d_ref):
    t_ref[...] = (jnp.dot(ea_ref[...], w_ref[..., D:, :],
                          preferred_element_type=jnp.float32) + b_ref[...])
    i = pl.program_id(0)
    sl = pl.ds(i * _PREP_BE, _PREP_BE)
    s_ref[sl] = ei_ref[0, sl]
    d_ref[sl] = ei_ref[1, sl]


def _upd_body(x_ref, a_ref, w_ref, b_ref, o_ref):
    agg = a_ref[0] + a_ref[1]
    h = (jnp.dot(x_ref[...], w_ref[..., :D, :],
                 preferred_element_type=jnp.float32)
         + jnp.dot(agg, w_ref[..., D:, :],
                   preferred_element_type=jnp.float32)
         + b_ref[...])
    o_ref[...] = jnp.maximum(h, 0.0) + x_ref[...]


def _tc_y(x, wm):
    bn = 2000
    return pl.pallas_call(
        _y_body,
        grid=(N // bn,),
        in_specs=[pl.BlockSpec((bn, D), lambda i: (i, 0)),
                  pl.BlockSpec((D + DE, H), lambda i: (0, 0))],
        out_specs=pl.BlockSpec((bn, H), lambda i: (i, 0)),
        out_shape=jax.ShapeDtypeStruct((N, H), jnp.float32),
    )(x, wm)


_PREP_BE = 6400


def _tc_prep(ea, wm, bm, ei):
    be = _PREP_BE
    return pl.pallas_call(
        _prep_body,
        grid=(E // be,),
        in_specs=[pl.BlockSpec((be, DE), lambda i: (i, 0)),
                  pl.BlockSpec((D + DE, H), lambda i: (0, 0)),
                  pl.BlockSpec((1, H), lambda i: (0, 0)),
                  pl.BlockSpec((2, E), lambda i: (0, 0))],
        out_specs=[pl.BlockSpec((be, H), lambda i: (i, 0)),
                   pl.BlockSpec((E,), lambda i: (0,)),
                   pl.BlockSpec((E,), lambda i: (0,))],
        out_shape=[jax.ShapeDtypeStruct((E, H), jnp.float32),
                   jax.ShapeDtypeStruct((E,), jnp.int32),
                   jax.ShapeDtypeStruct((E,), jnp.int32)],
    )(ea, wm, bm, ei)


def _tc_update(x, agg_p, wu, bu):
    bn = 2000
    return pl.pallas_call(
        _upd_body,
        grid=(N // bn,),
        in_specs=[pl.BlockSpec((bn, D), lambda i: (i, 0)),
                  pl.BlockSpec((2, bn, H), lambda i: (0, i, 0)),
                  pl.BlockSpec((D + H, D), lambda i: (0, 0)),
                  pl.BlockSpec((1, D), lambda i: (0, 0))],
        out_specs=pl.BlockSpec((bn, D), lambda i: (i, 0)),
        out_shape=jax.ShapeDtypeStruct((N, D), jnp.float32),
    )(x, agg_p, wu, bu)


def _sc_edges(src, dst, t, y):
    mesh = plsc.VectorSubcoreMesh(core_axis_name="c", subcore_axis_name="s")

    scratch = (
        [pltpu.VMEM((CHUNK,), jnp.int32) for _ in range(2 * NBUF)]
        + [pltpu.VMEM((NBUF * CHUNK, H), jnp.float32)]
        + [pltpu.SemaphoreType.DMA for _ in range(3 * NBUF)]
        + [pltpu.VMEM_SHARED((N, H), jnp.float32)]
    )

    @functools.partial(
        pl.kernel,
        mesh=mesh,
        out_type=jax.ShapeDtypeStruct((NC, N, H), jnp.float32),
        scratch_types=scratch,
    )
    def k(src_hbm, dst_hbm, t_hbm, y_hbm, out_hbm, *rest):
        idx_s = rest[0:NBUF]
        idx_d = rest[NBUF:2 * NBUF]
        mbuf = rest[2 * NBUF]
        lsem = rest[2 * NBUF + 1:2 * NBUF + 1 + NBUF]
        gsem = rest[2 * NBUF + 1 + NBUF:2 * NBUF + 1 + 2 * NBUF]
        ssem = rest[2 * NBUF + 1 + 2 * NBUF:2 * NBUF + 1 + 3 * NBUF]
        acc = rest[-1]

        cid = lax.axis_index("c")
        sid = lax.axis_index("s")
        wid = cid * NS + sid
        ebase = wid * EPW
        rbase = sid * RPT

        def msl(j):
            return mbuf.at[pl.ds(j * CHUNK, CHUNK)]

        @plsc.parallel_loop(0, NBUF * CHUNK, unroll=4)
        def _(i):
            for q in range(H // 16):
                mbuf[i, pl.ds(q * 16, 16)] = jnp.zeros((16,), jnp.float32)
        pltpu.sync_copy(mbuf.at[pl.ds(0, NBUF * CHUNK)],
                        acc.at[pl.ds(rbase, NBUF * CHUNK)])
        pltpu.sync_copy(mbuf.at[pl.ds(0, RPT - NBUF * CHUNK)],
                        acc.at[pl.ds(rbase + NBUF * CHUNK,
                                     RPT - NBUF * CHUNK)])

        @pl.when(sid == 0)
        def _():
            pltpu.sync_copy(mbuf.at[pl.ds(0, RTAIL)],
                            acc.at[pl.ds(NS * RPT, RTAIL)])
        plsc.subcore_barrier()

        def issue_loads(c, j):
            base = ebase + c * CHUNK
            pltpu.async_copy(src_hbm.at[pl.ds(base, CHUNK)], idx_s[j], lsem[j])
            pltpu.async_copy(dst_hbm.at[pl.ds(base, CHUNK)], idx_d[j], lsem[j])
            pltpu.async_copy(t_hbm.at[pl.ds(base, CHUNK)], msl(j), lsem[j])

        def wait_loads(c, j):
            base = ebase + c * CHUNK
            pltpu.make_async_copy(src_hbm.at[pl.ds(base, CHUNK)], idx_s[j],
                                  lsem[j]).wait()
            pltpu.make_async_copy(dst_hbm.at[pl.ds(base, CHUNK)], idx_d[j],
                                  lsem[j]).wait()
            pltpu.make_async_copy(t_hbm.at[pl.ds(base, CHUNK)], msl(j),
                                  lsem[j]).wait()

        def relu_buf(j):
            br = j * CHUNK

            @plsc.parallel_loop(0, CHUNK, unroll=4)
            def _(e):
                for q in range(H // 16):
                    sl = pl.ds(q * 16, 16)
                    mbuf[br + e, sl] = jnp.maximum(mbuf[br + e, sl], 0.0)

        for j in range(NBUF):
            issue_loads(j, j)

        def group(i, _):
            gathers = []
            for j in range(NBUF):
                wait_loads(i * NBUF + j, j)
                gathers.append(
                    pltpu.async_copy(y_hbm.at[idx_s[j]], msl(j), gsem[j],
                                     add=True))
            scatters = []
            for j in range(NBUF):
                gathers[j].wait()
                relu_buf(j)
                scatters.append(
                    pltpu.async_copy(msl(j), acc.at[idx_d[j]], ssem[j],
                                     add=True))
            for j in range(NBUF):
                scatters[j].wait()

                @pl.when(i < GROUPS - 1)
                def _(j=j):
                    issue_loads((i + 1) * NBUF + j, j)

                if j == 0:
                    @pl.when(i == GROUPS - 1)
                    def _():
                        issue_loads(NCHUNK - 1, 0)
            return 0
        lax.fori_loop(0, GROUPS, group, 0)

        wait_loads(NCHUNK - 1, 0)
        pltpu.async_copy(y_hbm.at[idx_s[0]], msl(0), gsem[0], add=True).wait()
        relu_buf(0)
        pltpu.async_copy(msl(0), acc.at[idx_d[0]], ssem[0], add=True).wait()

        plsc.subcore_barrier()
        pltpu.sync_copy(acc.at[pl.ds(rbase, RPT)],
                        out_hbm.at[cid, pl.ds(rbase, RPT)])

        @pl.when(sid == 0)
        def _():
            pltpu.sync_copy(acc.at[pl.ds(NS * RPT, RTAIL)],
                            out_hbm.at[cid, pl.ds(NS * RPT, RTAIL)])

    return k(src, dst, t, y)


def kernel(x, edge_index, edge_attr, W_msg, b_msg, W_upd, b_upd):
    bm = b_msg.reshape(1, H)
    bu = b_upd.reshape(1, D)

    y = _tc_y(x, W_msg)
    t, src, dst = _tc_prep(edge_attr, W_msg, bm, edge_index)
    agg_p = _sc_edges(src, dst, t, y)
    return _tc_update(x, agg_p, W_upd, bu)
